# baseline (speedup 1.0000x reference)
import functools

import jax
import jax.numpy as jnp
import numpy as np
from jax import lax
from jax.experimental import pallas as pl
from jax.experimental.pallas import tpu as pltpu
from jax.experimental.pallas import tpu_sc as plsc

DIM = 17
D2 = DIM * DIM
D3 = DIM * DIM * DIM
NLUT = 2 * D3
B = 16
PIX = 512 * 512
CHUNK = 8192
NWORKERS = 32
CHUNKS_PER_BATCH = PIX // CHUNK
TOTAL_CHUNKS = B * CHUNKS_PER_BATCH
CHUNKS_PER_WORKER = TOTAL_CHUNKS // NWORKERS

_RSCALE = np.float32((DIM - 1) / 1.000001)
_URSCALE = np.float32(1.0 / 1.000001)


def _lut_body(x_h, lr_h, lg_h, lb_h, out_h, lutR, lutG, lutB, xin, yout):
    pltpu.sync_copy(lr_h, lutR)
    pltpu.sync_copy(lg_h, lutG)
    pltpu.sync_copy(lb_h, lutB)

    cid = lax.axis_index("c")
    sid = lax.axis_index("s")
    wid = sid * 2 + cid

    def chunk_body(t, carry):
        b = t // CHUNKS_PER_BATCH
        q = (t % CHUNKS_PER_BATCH) * CHUNK
        pltpu.sync_copy(x_h.at[b, :, pl.ds(q, CHUNK)], xin)

        @plsc.parallel_loop(0, CHUNK // 16, unroll=2)
        def px(i):
            s = pl.ds(i * 16, 16)
            u = xin[0, s]
            r = xin[1, s]
            g = xin[2, s]
            bl = xin[3, s]
            rf = r * _RSCALE
            gf = g * _RSCALE
            bf = bl * _RSCALE
            ri = jnp.clip(rf.astype(jnp.int32), 0, DIM - 2)
            gi = jnp.clip(gf.astype(jnp.int32), 0, DIM - 2)
            bi = jnp.clip(bf.astype(jnp.int32), 0, DIM - 2)
            dr = rf - ri.astype(jnp.float32)
            dg = gf - gi.astype(jnp.float32)
            db = bf - bi.astype(jnp.float32)
            du = u * _URSCALE
            base = ri * D2 + gi * DIM + bi
            w_k = (1.0 - db, db)
            w_j = (1.0 - dg, dg)
            w_i = (1.0 - dr, dr)
            acc = [[None] * 3, [None] * 3]
            for di in (0, 1):
                for dj in (0, 1):
                    wij = w_i[di] * w_j[dj]
                    for dk in (0, 1):
                        wijk = wij * w_k[dk]
                        idx0 = base + (di * D2 + dj * DIM + dk)
                        idx1 = idx0 + D3
                        for p, idx in ((0, idx0), (1, idx1)):
                            for c, lut in enumerate((lutR, lutG, lutB)):
                                v = wijk * plsc.load_gather(lut, [idx])
                                a = acc[p][c]
                                acc[p][c] = v if a is None else a + v
            omu = 1.0 - du
            yout[0, s] = omu * acc[0][0] + du * acc[1][0]
            yout[1, s] = omu * acc[0][1] + du * acc[1][1]
            yout[2, s] = omu * acc[0][2] + du * acc[1][2]
        pltpu.sync_copy(yout, out_h.at[b, :, pl.ds(q, CHUNK)])
        return carry

    lax.fori_loop(wid * CHUNKS_PER_WORKER, (wid + 1) * CHUNKS_PER_WORKER,
                  chunk_body, 0)


_mesh = plsc.VectorSubcoreMesh(core_axis_name="c", subcore_axis_name="s")

_lut_apply = functools.partial(
    pl.kernel,
    out_type=jax.ShapeDtypeStruct((B, 3, PIX), jnp.float32),
    mesh=_mesh,
    scratch_types=[
        pltpu.VMEM((NLUT,), jnp.float32),
        pltpu.VMEM((NLUT,), jnp.float32),
        pltpu.VMEM((NLUT,), jnp.float32),
        pltpu.VMEM((4, CHUNK), jnp.float32),
        pltpu.VMEM((3, CHUNK), jnp.float32),
    ],
    compiler_params=pltpu.CompilerParams(needs_layout_passes=False),
)(_lut_body)


@jax.jit
def kernel(x, LUT_en):
    x3 = x.reshape(B, 4, PIX)
    lutf = LUT_en.reshape(3, NLUT)
    out3 = _lut_apply(x3, lutf[0], lutf[1], lutf[2])
    return out3.reshape(B, 3, 512, 512)


# speedup vs baseline: 1.2589x; 1.2589x over previous
import functools

import jax
import jax.numpy as jnp
import numpy as np
from jax import lax
from jax.experimental import pallas as pl
from jax.experimental.pallas import tpu as pltpu
from jax.experimental.pallas import tpu_sc as plsc

DIM = 17
D2 = DIM * DIM
D3 = DIM * DIM * DIM
NLUT = 2 * D3
B = 16
W = 512
PIX = 512 * 512
CHUNK = 8192
ROWS_PER_CHUNK = CHUNK // W
NWORKERS = 32
CHUNKS_PER_BATCH = PIX // CHUNK
TOTAL_CHUNKS = B * CHUNKS_PER_BATCH
CHUNKS_PER_WORKER = TOTAL_CHUNKS // NWORKERS

_RSCALE = np.float32((DIM - 1) / 1.000001)
_URSCALE = np.float32(1.0 / 1.000001)


def _lut_body(x_h, lr_h, lg_h, lb_h, out_h, lutR, lutG, lutB, xin, yout):
    pltpu.sync_copy(lr_h, lutR)
    pltpu.sync_copy(lg_h, lutG)
    pltpu.sync_copy(lb_h, lutB)

    cid = lax.axis_index("c")
    sid = lax.axis_index("s")
    wid = sid * 2 + cid

    def chunk_body(t, carry):
        b = t // CHUNKS_PER_BATCH
        row0 = (t % CHUNKS_PER_BATCH) * ROWS_PER_CHUNK
        pltpu.sync_copy(x_h.at[b, :, pl.ds(row0, ROWS_PER_CHUNK), :], xin)

        @plsc.parallel_loop(0, CHUNK // 16, unroll=2)
        def px(i):
            rr = i >> 5
            s = pl.ds((i & 31) * 16, 16)
            u = xin[0, rr, s]
            r = xin[1, rr, s]
            g = xin[2, rr, s]
            bl = xin[3, rr, s]
            rf = r * _RSCALE
            gf = g * _RSCALE
            bf = bl * _RSCALE
            ri = jnp.clip(rf.astype(jnp.int32), 0, DIM - 2)
            gi = jnp.clip(gf.astype(jnp.int32), 0, DIM - 2)
            bi = jnp.clip(bf.astype(jnp.int32), 0, DIM - 2)
            dr = rf - ri.astype(jnp.float32)
            dg = gf - gi.astype(jnp.float32)
            db = bf - bi.astype(jnp.float32)
            du = u * _URSCALE
            base = ri * D2 + gi * DIM + bi
            w_k = (1.0 - db, db)
            w_j = (1.0 - dg, dg)
            w_i = (1.0 - dr, dr)
            acc = [[None] * 3, [None] * 3]
            for di in (0, 1):
                for dj in (0, 1):
                    wij = w_i[di] * w_j[dj]
                    for dk in (0, 1):
                        wijk = wij * w_k[dk]
                        idx0 = base + (di * D2 + dj * DIM + dk)
                        idx1 = idx0 + D3
                        for p, idx in ((0, idx0), (1, idx1)):
                            for c, lut in enumerate((lutR, lutG, lutB)):
                                v = wijk * plsc.load_gather(lut, [idx])
                                a = acc[p][c]
                                acc[p][c] = v if a is None else a + v
            omu = 1.0 - du
            yout[0, rr, s] = omu * acc[0][0] + du * acc[1][0]
            yout[1, rr, s] = omu * acc[0][1] + du * acc[1][1]
            yout[2, rr, s] = omu * acc[0][2] + du * acc[1][2]
        pltpu.sync_copy(yout, out_h.at[b, :, pl.ds(row0, ROWS_PER_CHUNK), :])
        return carry

    lax.fori_loop(wid * CHUNKS_PER_WORKER, (wid + 1) * CHUNKS_PER_WORKER,
                  chunk_body, 0)


_mesh = plsc.VectorSubcoreMesh(core_axis_name="c", subcore_axis_name="s")

_lut_apply = functools.partial(
    pl.kernel,
    out_type=jax.ShapeDtypeStruct((B, 3, W, W), jnp.float32),
    mesh=_mesh,
    scratch_types=[
        pltpu.VMEM((NLUT,), jnp.float32),
        pltpu.VMEM((NLUT,), jnp.float32),
        pltpu.VMEM((NLUT,), jnp.float32),
        pltpu.VMEM((4, ROWS_PER_CHUNK, W), jnp.float32),
        pltpu.VMEM((3, ROWS_PER_CHUNK, W), jnp.float32),
    ],
    compiler_params=pltpu.CompilerParams(needs_layout_passes=False),
)(_lut_body)


@jax.jit
def kernel(x, LUT_en):
    lutf = LUT_en.reshape(3, NLUT)
    return _lut_apply(x, lutf[0], lutf[1], lutf[2])


# speedup vs baseline: 1.3842x; 1.0994x over previous
import functools

import jax
import jax.numpy as jnp
import numpy as np
from jax import lax
from jax.experimental import pallas as pl
from jax.experimental.pallas import tpu as pltpu
from jax.experimental.pallas import tpu_sc as plsc

DIM = 17
D2 = DIM * DIM
D3 = DIM * DIM * DIM
NLUT = 2 * D3
B = 16
W = 512
PIX = 512 * 512
CHUNK = 8192
ROWS_PER_CHUNK = CHUNK // W
NWORKERS = 32
CHUNKS_PER_BATCH = PIX // CHUNK
TOTAL_CHUNKS = B * CHUNKS_PER_BATCH
CHUNKS_PER_WORKER = TOTAL_CHUNKS // NWORKERS

_RSCALE = np.float32((DIM - 1) / 1.000001)
_URSCALE = np.float32(1.0 / 1.000001)


def _lut_body(x_h, lr_h, lg_h, lb_h, out_h, lutR, lutG, lutB, xin, yout):
    pltpu.sync_copy(lr_h, lutR)
    pltpu.sync_copy(lg_h, lutG)
    pltpu.sync_copy(lb_h, lutB)

    cid = lax.axis_index("c")
    sid = lax.axis_index("s")
    wid = sid * 2 + cid

    def chunk_body(t, carry):
        b = t // CHUNKS_PER_BATCH
        row0 = (t % CHUNKS_PER_BATCH) * ROWS_PER_CHUNK
        pltpu.sync_copy(x_h.at[b, :, pl.ds(row0, ROWS_PER_CHUNK), :], xin)

        def row_body(rr, carry2):
          @plsc.parallel_loop(0, W // 16, unroll=2)
          def px(j):
            s = pl.ds(j * 16, 16)
            u = xin[0, rr, s]
            r = xin[1, rr, s]
            g = xin[2, rr, s]
            bl = xin[3, rr, s]
            rf = r * _RSCALE
            gf = g * _RSCALE
            bf = bl * _RSCALE
            ri = jnp.clip(rf.astype(jnp.int32), 0, DIM - 2)
            gi = jnp.clip(gf.astype(jnp.int32), 0, DIM - 2)
            bi = jnp.clip(bf.astype(jnp.int32), 0, DIM - 2)
            dr = rf - ri.astype(jnp.float32)
            dg = gf - gi.astype(jnp.float32)
            db = bf - bi.astype(jnp.float32)
            du = u * _URSCALE
            base = ri * D2 + gi * DIM + bi
            w_k = (1.0 - db, db)
            w_j = (1.0 - dg, dg)
            w_i = (1.0 - dr, dr)
            acc = [[None] * 3, [None] * 3]
            for di in (0, 1):
                for dj in (0, 1):
                    wij = w_i[di] * w_j[dj]
                    for dk in (0, 1):
                        wijk = wij * w_k[dk]
                        idx0 = base + (di * D2 + dj * DIM + dk)
                        idx1 = idx0 + D3
                        for p, idx in ((0, idx0), (1, idx1)):
                            for c, lut in enumerate((lutR, lutG, lutB)):
                                v = wijk * plsc.load_gather(lut, [idx])
                                a = acc[p][c]
                                acc[p][c] = v if a is None else a + v
            omu = 1.0 - du
            yout[0, rr, s] = omu * acc[0][0] + du * acc[1][0]
            yout[1, rr, s] = omu * acc[0][1] + du * acc[1][1]
            yout[2, rr, s] = omu * acc[0][2] + du * acc[1][2]
          return carry2

        lax.fori_loop(0, ROWS_PER_CHUNK, row_body, 0)
        pltpu.sync_copy(yout, out_h.at[b, :, pl.ds(row0, ROWS_PER_CHUNK), :])
        return carry

    lax.fori_loop(wid * CHUNKS_PER_WORKER, (wid + 1) * CHUNKS_PER_WORKER,
                  chunk_body, 0)


_mesh = plsc.VectorSubcoreMesh(core_axis_name="c", subcore_axis_name="s")

_lut_apply = functools.partial(
    pl.kernel,
    out_type=jax.ShapeDtypeStruct((B, 3, W, W), jnp.float32),
    mesh=_mesh,
    scratch_types=[
        pltpu.VMEM((NLUT,), jnp.float32),
        pltpu.VMEM((NLUT,), jnp.float32),
        pltpu.VMEM((NLUT,), jnp.float32),
        pltpu.VMEM((4, ROWS_PER_CHUNK, W), jnp.float32),
        pltpu.VMEM((3, ROWS_PER_CHUNK, W), jnp.float32),
    ],
    compiler_params=pltpu.CompilerParams(needs_layout_passes=False),
)(_lut_body)


@jax.jit
def kernel(x, LUT_en):
    lutf = LUT_en.reshape(3, NLUT)
    return _lut_apply(x, lutf[0], lutf[1], lutf[2])
